# baseline (speedup 1.0000x reference)
import functools

import jax
import jax.numpy as jnp
from jax import lax
from jax.experimental import pallas as pl
from jax.experimental.pallas import tpu as pltpu
from jax.experimental.pallas import tpu_sc as plsc

NUM_HID = 64
PAIR_HID = 128
SEQ = 200
NC, NS, L = 2, 16, 16
NW = NC * NS
CHUNK = 128
NBUF = 2
POS_ROWS = 328


def _pos_body(out_ref):
    half = NUM_HID // 2
    r = lax.broadcasted_iota(jnp.int32, (POS_ROWS, NUM_HID), 0)
    pos = (r % SEQ).astype(jnp.float32)
    j = lax.broadcasted_iota(jnp.int32, (POS_ROWS, NUM_HID), 1)
    jj = jnp.where(j < half, j, j - half).astype(jnp.float32) / float(half)
    rate = jnp.exp(jj * (-jnp.log(10000.0)))
    ang = pos * rate
    out_ref[...] = jnp.where(j < half, jnp.sin(ang), jnp.cos(ang))


_pos_table = pl.pallas_call(
    _pos_body,
    out_shape=jax.ShapeDtypeStruct((POS_ROWS, NUM_HID), jnp.float32),
)


def _make_sc_kernel(total_rows):
    rows_per_w = total_rows // NW
    n_chunks = rows_per_w // CHUNK

    @functools.partial(
        pl.kernel,
        out_type=jax.ShapeDtypeStruct((total_rows, NUM_HID), jnp.float32),
        mesh=plsc.VectorSubcoreMesh(core_axis_name="c", subcore_axis_name="s"),
        scratch_types=(
            [pltpu.VMEM((CHUNK,), jnp.int32) for _ in range(NBUF)]
            + [pltpu.VMEM((CHUNK,), jnp.int32) for _ in range(NBUF)]
            + [pltpu.VMEM((CHUNK, PAIR_HID), jnp.float32) for _ in range(NBUF)]
            + [pltpu.VMEM((CHUNK, NUM_HID), jnp.float32) for _ in range(NBUF)]
            + [pltpu.VMEM((POS_ROWS, NUM_HID), jnp.float32)]
            + [pltpu.SemaphoreType.DMA for _ in range(4 * NBUF)]
        ),
        compiler_params=pltpu.CompilerParams(use_tc_tiling_on_sc=True,
                                             disable_bounds_checks=True),
    )
    def sc_kernel(x2_hbm, par_hbm, table2_hbm, pos_hbm, out_hbm, *scratch):
        pidx_v = scratch[:NBUF]
        par_v = scratch[NBUF:2 * NBUF]
        pair_v = scratch[2 * NBUF:3 * NBUF]
        rows_v = scratch[3 * NBUF:4 * NBUF]
        pos_v = scratch[4 * NBUF]
        isem = scratch[4 * NBUF + 1:4 * NBUF + 1 + NBUF]
        psem = scratch[4 * NBUF + 1 + NBUF:4 * NBUF + 1 + 2 * NBUF]
        gsem = scratch[4 * NBUF + 1 + 2 * NBUF:4 * NBUF + 1 + 3 * NBUF]
        osem = scratch[4 * NBUF + 1 + 3 * NBUF:]

        wid = lax.axis_index("s") * NC + lax.axis_index("c")
        base = wid * rows_per_w
        pltpu.sync_copy(pos_hbm, pos_v)

        def idx_start(b, c):
            off = base + c * CHUNK
            pltpu.async_copy(x2_hbm.at[pl.ds(off, CHUNK)], pidx_v[b], isem[b])

        def par_start(b, c):
            off = base + c * CHUNK
            pltpu.async_copy(par_hbm.at[pl.ds(off, CHUNK)], par_v[b], psem[b])

        def idx_wait(b):
            pltpu.make_async_copy(
                x2_hbm.at[pl.ds(0, CHUNK)], pidx_v[b], isem[b]).wait()

        def par_wait(b):
            pltpu.make_async_copy(
                par_hbm.at[pl.ds(0, CHUNK)], par_v[b], psem[b]).wait()

        def gather_start(b):
            pltpu.async_copy(table2_hbm.at[pidx_v[b]], pair_v[b], gsem[b])

        def gather_wait(b):
            pltpu.make_async_copy(
                table2_hbm.at[pidx_v[b]], pair_v[b], gsem[b]).wait()

        def out_desc(b, c):
            return pltpu.make_async_copy(
                rows_v[b], out_hbm.at[pl.ds(base + c * CHUNK, CHUNK)], osem[b])

        for b in range(NBUF):
            idx_start(b, b)
            par_start(b, b)
        for b in range(NBUF):
            idx_wait(b)
            gather_start(b)

        @pl.loop(0, n_chunks, step=NBUF)
        def _round(g):
            for b in range(NBUF):
                c = g + b
                gather_wait(b)
                nc = c + NBUF

                @pl.when(nc < n_chunks)
                def _pref():
                    idx_start(b, nc)

                par_wait(b)
                p0 = lax.rem(c * CHUNK, SEQ)

                @pl.loop(0, CHUNK // L)
                def _grp(g16):
                    parv = par_v[b][pl.ds(g16 * L, L)]
                    for l in range(L):
                        r = g16 * L + l
                        cbase = parv[l]
                        for k in range(NUM_HID // L):
                            pv = pair_v[b][r, pl.ds(cbase + k * L, L)]
                            po = pos_v[p0 + r, pl.ds(k * L, L)]
                            rows_v[b][r, pl.ds(k * L, L)] = pv + po

                out_desc(b, c).start()

                @pl.when(nc < n_chunks)
                def _next():
                    par_start(b, nc)
                    out_desc(b, c).wait()
                    idx_wait(b)
                    gather_start(b)

        for b in range(NBUF):
            out_desc(b, n_chunks - NBUF + b).wait()

    return sc_kernel


def kernel(x, table):
    b, s = x.shape
    total = b * s
    v = table.shape[0]
    pos = _pos_table()
    x_flat = x.reshape(-1)
    x2 = lax.shift_right_logical(x_flat, 1)
    par = lax.shift_left(lax.bitwise_and(x_flat, 1), 6)
    table2 = table.reshape(v // 2, 2 * NUM_HID)
    out = _make_sc_kernel(total)(x2, par, table2, pos)
    return out.reshape(b, s, NUM_HID)


# speedup vs baseline: 1.2173x; 1.2173x over previous
import functools

import jax
import jax.numpy as jnp
from jax import lax
from jax.experimental import pallas as pl
from jax.experimental.pallas import tpu as pltpu
from jax.experimental.pallas import tpu_sc as plsc

NUM_HID = 64
PAIR_HID = 128
SEQ = 200
NC, NS, L = 2, 16, 16
NW = NC * NS
CHUNK = 128
NBUF = 2
POS_ROWS = 328


def _pos_body(out_ref):
    half = NUM_HID // 2
    r = lax.broadcasted_iota(jnp.int32, (POS_ROWS, NUM_HID), 0)
    pos = (r % SEQ).astype(jnp.float32)
    j = lax.broadcasted_iota(jnp.int32, (POS_ROWS, NUM_HID), 1)
    jj = jnp.where(j < half, j, j - half).astype(jnp.float32) / float(half)
    rate = jnp.exp(jj * (-jnp.log(10000.0)))
    ang = pos * rate
    out_ref[...] = jnp.where(j < half, jnp.sin(ang), jnp.cos(ang))


_pos_table = pl.pallas_call(
    _pos_body,
    out_shape=jax.ShapeDtypeStruct((POS_ROWS, NUM_HID), jnp.float32),
)

PACK_NB = 2048
PACK_GRID = 245
H2 = PACK_NB * PACK_GRID


def _pack_body(t1_ref, t2_ref, out_ref):
    t1 = jnp.transpose(t1_ref[...], (1, 0))
    t2 = jnp.transpose(t2_ref[...], (1, 0))
    out_ref[...] = jnp.concatenate([t1, t2], axis=1)


def _pack_table(table_t):
    vocab = table_t.shape[1]
    last_blk = (vocab + PACK_NB - 1) // PACK_NB - 1
    return pl.pallas_call(
        _pack_body,
        grid=(PACK_GRID,),
        in_specs=[
            pl.BlockSpec((NUM_HID, PACK_NB), lambda i: (0, i)),
            pl.BlockSpec(
                (NUM_HID, PACK_NB),
                lambda i: (0, jnp.minimum(i + PACK_GRID, last_blk))),
        ],
        out_specs=pl.BlockSpec((PACK_NB, 2 * NUM_HID), lambda i: (i, 0)),
        out_shape=jax.ShapeDtypeStruct((H2, 2 * NUM_HID), jnp.float32),
    )(table_t, table_t)


def _make_sc_kernel(total_rows):
    rows_per_w = total_rows // NW
    n_chunks = rows_per_w // CHUNK

    @functools.partial(
        pl.kernel,
        out_type=jax.ShapeDtypeStruct((total_rows, NUM_HID), jnp.float32),
        mesh=plsc.VectorSubcoreMesh(core_axis_name="c", subcore_axis_name="s"),
        scratch_types=(
            [pltpu.VMEM((CHUNK,), jnp.int32) for _ in range(NBUF)]
            + [pltpu.VMEM((CHUNK,), jnp.int32) for _ in range(NBUF)]
            + [pltpu.VMEM((CHUNK, PAIR_HID), jnp.float32) for _ in range(NBUF)]
            + [pltpu.VMEM((CHUNK, NUM_HID), jnp.float32) for _ in range(NBUF)]
            + [pltpu.VMEM((POS_ROWS, NUM_HID), jnp.float32)]
            + [pltpu.SemaphoreType.DMA for _ in range(4 * NBUF)]
        ),
        compiler_params=pltpu.CompilerParams(use_tc_tiling_on_sc=True,
                                             disable_bounds_checks=True),
    )
    def sc_kernel(x2_hbm, par_hbm, table2_hbm, pos_hbm, out_hbm, *scratch):
        pidx_v = scratch[:NBUF]
        par_v = scratch[NBUF:2 * NBUF]
        pair_v = scratch[2 * NBUF:3 * NBUF]
        rows_v = scratch[3 * NBUF:4 * NBUF]
        pos_v = scratch[4 * NBUF]
        isem = scratch[4 * NBUF + 1:4 * NBUF + 1 + NBUF]
        psem = scratch[4 * NBUF + 1 + NBUF:4 * NBUF + 1 + 2 * NBUF]
        gsem = scratch[4 * NBUF + 1 + 2 * NBUF:4 * NBUF + 1 + 3 * NBUF]
        osem = scratch[4 * NBUF + 1 + 3 * NBUF:]

        wid = lax.axis_index("s") * NC + lax.axis_index("c")
        base = wid * rows_per_w
        pltpu.sync_copy(pos_hbm, pos_v)

        def idx_start(b, c):
            off = base + c * CHUNK
            pltpu.async_copy(x2_hbm.at[pl.ds(off, CHUNK)], pidx_v[b], isem[b])

        def par_start(b, c):
            off = base + c * CHUNK
            pltpu.async_copy(par_hbm.at[pl.ds(off, CHUNK)], par_v[b], psem[b])

        def idx_wait(b):
            pltpu.make_async_copy(
                x2_hbm.at[pl.ds(0, CHUNK)], pidx_v[b], isem[b]).wait()

        def par_wait(b):
            pltpu.make_async_copy(
                par_hbm.at[pl.ds(0, CHUNK)], par_v[b], psem[b]).wait()

        def gather_start(b):
            pltpu.async_copy(table2_hbm.at[pidx_v[b]], pair_v[b], gsem[b])

        def gather_wait(b):
            pltpu.make_async_copy(
                table2_hbm.at[pidx_v[b]], pair_v[b], gsem[b]).wait()

        def out_desc(b, c):
            return pltpu.make_async_copy(
                rows_v[b], out_hbm.at[pl.ds(base + c * CHUNK, CHUNK)], osem[b])

        for b in range(NBUF):
            idx_start(b, b)
            par_start(b, b)
        for b in range(NBUF):
            idx_wait(b)
            gather_start(b)

        @pl.loop(0, n_chunks, step=NBUF)
        def _round(g):
            for b in range(NBUF):
                c = g + b
                gather_wait(b)
                nc = c + NBUF

                @pl.when(nc < n_chunks)
                def _pref():
                    idx_start(b, nc)

                par_wait(b)
                p0 = lax.rem(c * CHUNK, SEQ)

                @pl.loop(0, CHUNK // L)
                def _grp(g16):
                    parv = par_v[b][pl.ds(g16 * L, L)]
                    for l in range(L):
                        r = g16 * L + l
                        cbase = parv[l]
                        for k in range(NUM_HID // L):
                            pv = pair_v[b][r, pl.ds(cbase + k * L, L)]
                            po = pos_v[p0 + r, pl.ds(k * L, L)]
                            rows_v[b][r, pl.ds(k * L, L)] = pv + po

                out_desc(b, c).start()

                @pl.when(nc < n_chunks)
                def _next():
                    par_start(b, nc)
                    out_desc(b, c).wait()
                    idx_wait(b)
                    gather_start(b)

        for b in range(NBUF):
            out_desc(b, n_chunks - NBUF + b).wait()

    return sc_kernel


def kernel(x, table):
    b, s = x.shape
    total = b * s
    v = table.shape[0]
    del v
    pos = _pos_table()
    x_flat = x.reshape(-1)
    top = x_flat < H2
    x2 = jnp.where(top, x_flat, x_flat - H2)
    par = jnp.where(top, 0, NUM_HID).astype(jnp.int32)
    table2 = _pack_table(table.T)
    out = _make_sc_kernel(total)(x2, par, table2, pos)
    return out.reshape(b, s, NUM_HID)


# speedup vs baseline: 1.5529x; 1.2757x over previous
import functools

import jax
import jax.numpy as jnp
from jax import lax
from jax.experimental import pallas as pl
from jax.experimental.pallas import tpu as pltpu
from jax.experimental.pallas import tpu_sc as plsc

NUM_HID = 64
PAIR_HID = 128
SEQ = 200
NC, NS, L = 2, 16, 16
NW = NC * NS
CHUNK = 128
NBUF = 2
POS_ROWS = 328


def _pos_body(out_ref):
    half = NUM_HID // 2
    r = lax.broadcasted_iota(jnp.int32, (POS_ROWS, NUM_HID), 0)
    pos = (r % SEQ).astype(jnp.float32)
    j = lax.broadcasted_iota(jnp.int32, (POS_ROWS, NUM_HID), 1)
    jj = jnp.where(j < half, j, j - half).astype(jnp.float32) / float(half)
    rate = jnp.exp(jj * (-jnp.log(10000.0)))
    ang = pos * rate
    out_ref[...] = jnp.where(j < half, jnp.sin(ang), jnp.cos(ang))


_pos_table = pl.pallas_call(
    _pos_body,
    out_shape=jax.ShapeDtypeStruct((POS_ROWS, NUM_HID), jnp.float32),
)

PACK_NB = 2048
PACK_GRID = 245
H2 = PACK_NB * PACK_GRID


def _pack_body(t1_ref, t2_ref, out_ref):
    t1 = jnp.transpose(t1_ref[...], (1, 0))
    t2 = jnp.transpose(t2_ref[...], (1, 0))
    out_ref[...] = jnp.concatenate([t1, t2], axis=1)


def _pack_table(table_t):
    vocab = table_t.shape[1]
    last_blk = (vocab + PACK_NB - 1) // PACK_NB - 1
    return pl.pallas_call(
        _pack_body,
        grid=(PACK_GRID,),
        in_specs=[
            pl.BlockSpec((NUM_HID, PACK_NB), lambda i: (0, i)),
            pl.BlockSpec(
                (NUM_HID, PACK_NB),
                lambda i: (0, jnp.minimum(i + PACK_GRID, last_blk))),
        ],
        out_specs=pl.BlockSpec((PACK_NB, 2 * NUM_HID), lambda i: (i, 0)),
        out_shape=jax.ShapeDtypeStruct((H2, 2 * NUM_HID), jnp.float32),
    )(table_t, table_t)


def _make_sc_kernel(total_rows):
    rows_per_w = total_rows // NW
    n_chunks = rows_per_w // CHUNK

    @functools.partial(
        pl.kernel,
        out_type=jax.ShapeDtypeStruct((total_rows, NUM_HID), jnp.float32),
        mesh=plsc.VectorSubcoreMesh(core_axis_name="c", subcore_axis_name="s"),
        scratch_types=(
            [pltpu.VMEM((CHUNK,), jnp.int32) for _ in range(NBUF)]
            + [pltpu.VMEM((CHUNK,), jnp.int32) for _ in range(NBUF)]
            + [pltpu.VMEM((CHUNK, PAIR_HID), jnp.float32) for _ in range(NBUF)]
            + [pltpu.VMEM((CHUNK, NUM_HID), jnp.float32) for _ in range(NBUF)]
            + [pltpu.VMEM((POS_ROWS, NUM_HID), jnp.float32)]
            + [pltpu.SemaphoreType.DMA for _ in range(4 * NBUF)]
        ),
        compiler_params=pltpu.CompilerParams(use_tc_tiling_on_sc=True,
                                             disable_bounds_checks=True),
    )
    def sc_kernel(x2_hbm, par_hbm, table2_hbm, pos_hbm, out_hbm, *scratch):
        pidx_v = scratch[:NBUF]
        par_v = scratch[NBUF:2 * NBUF]
        pair_v = scratch[2 * NBUF:3 * NBUF]
        rows_v = scratch[3 * NBUF:4 * NBUF]
        pos_v = scratch[4 * NBUF]
        isem = scratch[4 * NBUF + 1:4 * NBUF + 1 + NBUF]
        psem = scratch[4 * NBUF + 1 + NBUF:4 * NBUF + 1 + 2 * NBUF]
        gsem = scratch[4 * NBUF + 1 + 2 * NBUF:4 * NBUF + 1 + 3 * NBUF]
        osem = scratch[4 * NBUF + 1 + 3 * NBUF:]

        wid = lax.axis_index("s") * NC + lax.axis_index("c")
        base = wid * rows_per_w
        pltpu.sync_copy(pos_hbm, pos_v)

        def idx_start(b, c):
            off = base + c * CHUNK
            pltpu.async_copy(x2_hbm.at[pl.ds(off, CHUNK)], pidx_v[b], isem[b])

        def par_start(b, c):
            off = base + c * CHUNK
            pltpu.async_copy(par_hbm.at[pl.ds(off, CHUNK)], par_v[b], psem[b])

        def idx_wait(b):
            pltpu.make_async_copy(
                x2_hbm.at[pl.ds(0, CHUNK)], pidx_v[b], isem[b]).wait()

        def par_wait(b):
            pltpu.make_async_copy(
                par_hbm.at[pl.ds(0, CHUNK)], par_v[b], psem[b]).wait()

        def gather_start(b):
            pltpu.async_copy(table2_hbm.at[pidx_v[b]], pair_v[b], gsem[b])

        def gather_wait(b):
            pltpu.make_async_copy(
                table2_hbm.at[pidx_v[b]], pair_v[b], gsem[b]).wait()

        def out_desc(b, c):
            return pltpu.make_async_copy(
                rows_v[b], out_hbm.at[pl.ds(base + c * CHUNK, CHUNK)], osem[b])

        for b in range(NBUF):
            idx_start(b, b)
            par_start(b, b)
        for b in range(NBUF):
            idx_wait(b)
            gather_start(b)

        @pl.loop(0, n_chunks, step=NBUF)
        def _round(g):
            for b in range(NBUF):
                c = g + b
                gather_wait(b)
                nc = c + NBUF

                @pl.when(nc < n_chunks)
                def _pref():
                    idx_start(b, nc)

                par_wait(b)
                p0 = lax.rem(c * CHUNK, SEQ)

                @plsc.parallel_loop(0, CHUNK // L, unroll=2)
                def _grp(g16):
                    parv = par_v[b][pl.ds(g16 * L, L)]
                    for l in range(L):
                        r = g16 * L + l
                        cbase = parv[l]
                        for k in range(NUM_HID // L):
                            pv = pair_v[b][r, pl.ds(cbase + k * L, L)]
                            po = pos_v[p0 + r, pl.ds(k * L, L)]
                            rows_v[b][r, pl.ds(k * L, L)] = pv + po

                out_desc(b, c).start()

                @pl.when(nc < n_chunks)
                def _next():
                    par_start(b, nc)
                    out_desc(b, c).wait()
                    idx_wait(b)
                    gather_start(b)

        for b in range(NBUF):
            out_desc(b, n_chunks - NBUF + b).wait()

    return sc_kernel


def kernel(x, table):
    b, s = x.shape
    total = b * s
    v = table.shape[0]
    del v
    pos = _pos_table()
    x_flat = x.reshape(-1)
    top = x_flat < H2
    x2 = jnp.where(top, x_flat, x_flat - H2)
    par = jnp.where(top, 0, NUM_HID).astype(jnp.int32)
    table2 = _pack_table(table.T)
    out = _make_sc_kernel(total)(x2, par, table2, pos)
    return out.reshape(b, s, NUM_HID)
